# baseline (speedup 1.0000x reference)
import jax
import jax.numpy as jnp
from jax import lax
from jax.experimental import pallas as pl
from jax.experimental.pallas import tpu as pltpu
from jax.experimental.pallas import tpu_sc as plsc

_ROUTES = 8
_WIN = 256


def _build_sc_gather(B, H, W, C, RW):
    info = plsc.get_sparse_core_info()
    NC, NS = info.num_cores, info.num_subcores
    NW = NC * NS
    assert B % NW == 0, (B, NW)
    BPW = B // NW
    NS_H = 8
    HH = H // NS_H
    NU = BPW * NS_H
    def c_window(r):
        return lax.shift_left(lax.shift_right_logical(r * RW, 7), 7)

    mesh = plsc.VectorSubcoreMesh(core_axis_name="c", subcore_axis_name="s")

    def body(table_hbm, logits_hbm, out_hbm, logits_v, wide, outb, rsem, wsem):
        wid = lax.axis_index("s") * NC + lax.axis_index("c")
        b0 = wid * BPW
        lbase = pl.multiple_of(
            lax.shift_left(lax.shift_right_logical(b0 * 16, 7), 7), 128)
        off = b0 * 16 - lbase
        pltpu.sync_copy(logits_hbm.at[pl.ds(lbase, 128)], logits_v)

        def route_of(u):
            v = logits_v[pl.ds(off + lax.shift_right_logical(u, 3) * 16, 16)]
            best_v = v[0]
            best_i = jnp.int32(0)
            for t in range(1, _ROUTES):
                x = v[t]
                take = x > best_v
                best_i = jnp.where(take, jnp.int32(t), best_i)
                best_v = jnp.where(take, x, best_v)
            return best_i

        def start_read(u):
            r = route_of(u)
            cs = pl.multiple_of(c_window(r), 128)
            a = r * RW - cs
            b = b0 + lax.shift_right_logical(u, 3)
            s = lax.bitwise_and(u, 3)
            h0 = lax.bitwise_and(u, 7) * HH

            @pl.when(a <= 32)
            def _():
                pltpu.make_async_copy(
                    table_hbm.at[b, pl.ds(h0, HH), :, pl.ds(cs, 128)],
                    wide.at[s, :, :, pl.ds(0, 128)], rsem.at[s]).start()

            @pl.when(a > 32)
            def _():
                pltpu.make_async_copy(
                    table_hbm.at[b, pl.ds(h0, HH), :, pl.ds(cs, _WIN)],
                    wide.at[s], rsem.at[s]).start()

        def unit(u, _):
            s = lax.bitwise_and(u, 3)
            b = b0 + lax.shift_right_logical(u, 3)
            h0 = lax.bitwise_and(u, 7) * HH
            dst_hbm = out_hbm.at[b, pl.ds(h0, HH)]
            r = route_of(u)
            a = pl.multiple_of(r * RW - pl.multiple_of(c_window(r), 128), 16)

            @pl.when(a <= 32)
            def _():
                pltpu.make_async_copy(
                    table_hbm.at[b, pl.ds(h0, HH), :, pl.ds(0, 128)],
                    wide.at[s, :, :, pl.ds(0, 128)], rsem.at[s]).wait()

            @pl.when(a > 32)
            def _():
                pltpu.make_async_copy(
                    table_hbm.at[b, pl.ds(h0, HH), :, pl.ds(0, _WIN)],
                    wide.at[s], rsem.at[s]).wait()
            src = wide.at[s]
            dst = outb.at[s]

            @plsc.parallel_loop(0, HH * W, step=1, unroll=2)
            def _row(i):
                h = lax.shift_right_logical(i, 4)
                w = lax.bitwise_and(i, W - 1)
                for k in range(RW // 16):
                    dst[h, w, pl.ds(16 * k, 16)] = (
                        src[h, w, pl.ds(a + 16 * k, 16)])

            pltpu.make_async_copy(dst, dst_hbm, wsem.at[s]).start()

            @pl.when(u + 3 < NU)
            def _():
                @pl.when(u >= 1)
                def _():
                    sp = lax.bitwise_and(u - 1, 3)
                    pltpu.make_async_copy(
                        outb.at[sp], out_hbm.at[b0, pl.ds(0, HH)],
                        wsem.at[sp]).wait()
                start_read(u + 3)
            return 0

        start_read(0)
        start_read(1)
        start_read(2)
        lax.fori_loop(0, NU, unit, 0)
        for un in range(NU - 4, NU):
            s = un & 3
            pltpu.make_async_copy(
                outb.at[s], out_hbm.at[b0, pl.ds(0, HH)], wsem.at[s]).wait()

    return pl.kernel(
        body,
        out_type=jax.ShapeDtypeStruct((B, H, W, RW), jnp.float32),
        mesh=mesh,
        compiler_params=pltpu.CompilerParams(use_tc_tiling_on_sc=True),
        scratch_types=[
            pltpu.VMEM((128,), jnp.float32),
            pltpu.VMEM((4, HH, W, _WIN), jnp.float32),
            pltpu.VMEM((4, HH, W, RW), jnp.float32),
            pltpu.SemaphoreType.DMA((4,)),
            pltpu.SemaphoreType.DMA((4,)),
        ],
    )


def kernel(inputs, routing_inputs):
    B, H, W, C = inputs.shape
    RW = C // _ROUTES
    logits = jnp.concatenate(
        [routing_inputs,
         jnp.full((B, 16 - _ROUTES), -jnp.inf, jnp.float32)], axis=1
    ).reshape(-1)
    return _build_sc_gather(B, H, W, C, RW)(inputs, logits)


# speedup vs baseline: 1.0390x; 1.0390x over previous
import jax
import jax.numpy as jnp
from jax import lax
from jax.experimental import pallas as pl
from jax.experimental.pallas import tpu as pltpu
from jax.experimental.pallas import tpu_sc as plsc

_ROUTES = 8
_WIN = 256


def _build_sc_gather(B, H, W, C, RW):
    info = plsc.get_sparse_core_info()
    NC, NS = info.num_cores, info.num_subcores
    NW = NC * NS
    assert B % NW == 0, (B, NW)
    BPW = B // NW
    NS_H = 4
    HH = H // NS_H
    NU = BPW * NS_H
    def c_window(r):
        return lax.shift_left(lax.shift_right_logical(r * RW, 7), 7)

    mesh = plsc.VectorSubcoreMesh(core_axis_name="c", subcore_axis_name="s")

    def body(table_hbm, logits_hbm, out_hbm, logits_v, wide, outb, rsem, wsem):
        wid = lax.axis_index("s") * NC + lax.axis_index("c")
        b0 = wid * BPW
        lbase = pl.multiple_of(
            lax.shift_left(lax.shift_right_logical(b0 * 16, 7), 7), 128)
        off = b0 * 16 - lbase
        pltpu.sync_copy(logits_hbm.at[pl.ds(lbase, 128)], logits_v)

        def route_of(u):
            v = logits_v[pl.ds(off + lax.shift_right_logical(u, 2) * 16, 16)]
            best_v = v[0]
            best_i = jnp.int32(0)
            for t in range(1, _ROUTES):
                x = v[t]
                take = x > best_v
                best_i = jnp.where(take, jnp.int32(t), best_i)
                best_v = jnp.where(take, x, best_v)
            return best_i

        def start_read(u):
            r = route_of(u)
            cs = pl.multiple_of(c_window(r), 128)
            a = r * RW - cs
            b = b0 + lax.shift_right_logical(u, 2)
            s = lax.bitwise_and(u, 3)
            h0 = s * HH

            @pl.when(a <= 32)
            def _():
                pltpu.make_async_copy(
                    table_hbm.at[b, pl.ds(h0, HH), :, pl.ds(cs, 128)],
                    wide.at[s, :, :, pl.ds(0, 128)], rsem.at[s]).start()

            @pl.when(a > 32)
            def _():
                pltpu.make_async_copy(
                    table_hbm.at[b, pl.ds(h0, HH), :, pl.ds(cs, _WIN)],
                    wide.at[s], rsem.at[s]).start()

        def unit(u, _):
            s = lax.bitwise_and(u, 3)
            b = b0 + lax.shift_right_logical(u, 2)
            h0 = s * HH
            dst_hbm = out_hbm.at[b, pl.ds(h0, HH)]
            r = route_of(u)
            a = pl.multiple_of(r * RW - pl.multiple_of(c_window(r), 128), 16)

            @pl.when(a <= 32)
            def _():
                pltpu.make_async_copy(
                    table_hbm.at[b, pl.ds(h0, HH), :, pl.ds(0, 128)],
                    wide.at[s, :, :, pl.ds(0, 128)], rsem.at[s]).wait()

            @pl.when(a > 32)
            def _():
                pltpu.make_async_copy(
                    table_hbm.at[b, pl.ds(h0, HH), :, pl.ds(0, _WIN)],
                    wide.at[s], rsem.at[s]).wait()
            src = wide.at[s]
            dst = outb.at[s]

            @pl.when(u + 3 < NU)
            def _():
                @pl.when(u >= 1)
                def _():
                    sp = lax.bitwise_and(u - 1, 3)
                    pltpu.make_async_copy(
                        outb.at[sp], out_hbm.at[b0, pl.ds(0, HH)],
                        wsem.at[sp]).wait()
                start_read(u + 3)

            @plsc.parallel_loop(0, HH * W, step=1, unroll=2)
            def _row(i):
                h = lax.shift_right_logical(i, 4)
                w = lax.bitwise_and(i, W - 1)
                for k in range(RW // 16):
                    dst[h, w, pl.ds(16 * k, 16)] = (
                        src[h, w, pl.ds(a + 16 * k, 16)])

            pltpu.make_async_copy(dst, dst_hbm, wsem.at[s]).start()
            return 0

        start_read(0)
        start_read(1)
        start_read(2)
        lax.fori_loop(0, NU, unit, 0)
        for un in range(NU - 4, NU):
            s = un & 3
            pltpu.make_async_copy(
                outb.at[s], out_hbm.at[b0, pl.ds(0, HH)], wsem.at[s]).wait()

    return pl.kernel(
        body,
        out_type=jax.ShapeDtypeStruct((B, H, W, RW), jnp.float32),
        mesh=mesh,
        compiler_params=pltpu.CompilerParams(use_tc_tiling_on_sc=True),
        scratch_types=[
            pltpu.VMEM((128,), jnp.float32),
            pltpu.VMEM((4, HH, W, _WIN), jnp.float32),
            pltpu.VMEM((4, HH, W, RW), jnp.float32),
            pltpu.SemaphoreType.DMA((4,)),
            pltpu.SemaphoreType.DMA((4,)),
        ],
    )


def kernel(inputs, routing_inputs):
    B, H, W, C = inputs.shape
    RW = C // _ROUTES
    logits = jnp.concatenate(
        [routing_inputs,
         jnp.full((B, 16 - _ROUTES), -jnp.inf, jnp.float32)], axis=1
    ).reshape(-1)
    return _build_sc_gather(B, H, W, C, RW)(inputs, logits)
